# baseline (speedup 1.0000x reference)
import jax
import jax.numpy as jnp
from jax.experimental import pallas as pl
from jax.experimental.pallas import tpu as pltpu

_K = 32
_S = 250
_NDROP = 4
_BD = 64
_BH = 64


def _body(drop_ref, x_ref, o_ref):
    bd, bh = x_ref.shape[1], x_ref.shape[2]
    d0 = pl.program_id(1) * bd
    h0 = pl.program_id(2) * bh
    pd = (d0 + jax.lax.broadcasted_iota(jnp.int32, (1, bd, 1, 1), 1)) // _K
    ph = (h0 + jax.lax.broadcasted_iota(jnp.int32, (1, 1, bh, 1), 2)) // _K
    pw = jax.lax.broadcasted_iota(jnp.int32, (1, 1, 1, _S), 3) // _K
    mask = None
    for n in range(_NDROP):
        p = drop_ref[n]
        m = (pd == p // 64) & (ph == (p // 8) % 8) & (pw == p % 8)
        mask = m if mask is None else mask | m
    o_ref[...] = jnp.where(mask, 0.0, x_ref[...])


def kernel(x, drop_idx):
    B = x.shape[0]
    xs = x.reshape(B, _S, _S, _S)
    gd = (_S + _BD - 1) // _BD
    gh = (_S + _BH - 1) // _BH
    out = pl.pallas_call(
        _body,
        grid=(B, gd, gh),
        in_specs=[
            pl.BlockSpec(memory_space=pltpu.SMEM),
            pl.BlockSpec((1, _BD, _BH, _S), lambda b, i, j: (b, i, j, 0)),
        ],
        out_specs=pl.BlockSpec((1, _BD, _BH, _S), lambda b, i, j: (b, i, j, 0)),
        out_shape=jax.ShapeDtypeStruct((B, _S, _S, _S), jnp.float32),
        compiler_params=pltpu.CompilerParams(
            dimension_semantics=("parallel", "parallel", "parallel"),
        ),
    )(drop_idx.astype(jnp.int32), xs)
    return out.reshape(x.shape)


# speedup vs baseline: 1.0214x; 1.0214x over previous
import jax
import jax.numpy as jnp
from jax.experimental import pallas as pl
from jax.experimental.pallas import tpu as pltpu

_K = 32
_S = 250
_NDROP = 4
_BD = 16
_BH = 250


def _body(drop_ref, x_ref, o_ref):
    bd, bh = x_ref.shape[1], x_ref.shape[2]
    d0 = pl.program_id(1) * bd
    h0 = pl.program_id(2) * bh
    pd = (d0 + jax.lax.broadcasted_iota(jnp.int32, (1, bd, 1, 1), 1)) // _K
    ph = (h0 + jax.lax.broadcasted_iota(jnp.int32, (1, 1, bh, 1), 2)) // _K
    pw = jax.lax.broadcasted_iota(jnp.int32, (1, 1, 1, _S), 3) // _K
    mask = None
    for n in range(_NDROP):
        p = drop_ref[n]
        m = (pd == p // 64) & (ph == (p // 8) % 8) & (pw == p % 8)
        mask = m if mask is None else mask | m
    o_ref[...] = jnp.where(mask, 0.0, x_ref[...])


def kernel(x, drop_idx):
    B = x.shape[0]
    xs = x.reshape(B, _S, _S, _S)
    gd = (_S + _BD - 1) // _BD
    gh = (_S + _BH - 1) // _BH
    out = pl.pallas_call(
        _body,
        grid=(B, gd, gh),
        in_specs=[
            pl.BlockSpec(memory_space=pltpu.SMEM),
            pl.BlockSpec((1, _BD, _BH, _S), lambda b, i, j: (b, i, j, 0)),
        ],
        out_specs=pl.BlockSpec((1, _BD, _BH, _S), lambda b, i, j: (b, i, j, 0)),
        out_shape=jax.ShapeDtypeStruct((B, _S, _S, _S), jnp.float32),
        compiler_params=pltpu.CompilerParams(
            dimension_semantics=("parallel", "parallel", "parallel"),
        ),
    )(drop_idx.astype(jnp.int32), xs)
    return out.reshape(x.shape)
